# baseline (speedup 1.0000x reference)
import jax
import jax.numpy as jnp
from jax.experimental import pallas as pl
from jax.experimental.pallas import tpu as pltpu


def _plane_max_kernel(x_ref, o_ref):
    o_ref[...] = jnp.max(x_ref[...], axis=(0, 1))


def kernel(x):
    N, C, H, W = x.shape
    xt = jnp.transpose(x, (2, 3, 0, 1))

    BN = 32
    grid = (N // BN,)
    out2d = pl.pallas_call(
        _plane_max_kernel,
        out_shape=jax.ShapeDtypeStruct((N, C), x.dtype),
        grid=grid,
        in_specs=[pl.BlockSpec((H, W, BN, C), lambda i: (0, 0, i, 0))],
        out_specs=pl.BlockSpec((BN, C), lambda i: (i, 0)),
        compiler_params=pltpu.CompilerParams(
            dimension_semantics=("parallel",),
            vmem_limit_bytes=64 * 1024 * 1024,
        ),
    )(xt)
    return out2d.reshape(N, C, 1, 1)


# speedup vs baseline: 1.0643x; 1.0643x over previous
import jax
import jax.numpy as jnp
from jax.experimental import pallas as pl
from jax.experimental.pallas import tpu as pltpu


def _plane_max_kernel(x_ref, o_ref):
    m = jnp.max(x_ref[...], axis=(0, 1))
    o_ref[...] = m.reshape(o_ref.shape)


def kernel(x):
    N, C, H, W = x.shape
    xt = jnp.transpose(x, (2, 3, 0, 1))

    BN = 32
    R = C // 128
    grid = (N // BN,)
    out2d = pl.pallas_call(
        _plane_max_kernel,
        out_shape=jax.ShapeDtypeStruct((N * R, 128), x.dtype),
        grid=grid,
        in_specs=[pl.BlockSpec((H, W, BN, C), lambda i: (0, 0, i, 0))],
        out_specs=pl.BlockSpec((BN * R, 128), lambda i: (i, 0)),
        compiler_params=pltpu.CompilerParams(
            dimension_semantics=("parallel",),
            vmem_limit_bytes=64 * 1024 * 1024,
        ),
    )(xt)
    return out2d.reshape(N, C, 1, 1)
